# baseline (speedup 1.0000x reference)
import functools

import jax
import jax.numpy as jnp
from jax import lax
from jax.experimental import pallas as pl
from jax.experimental.pallas import tpu as pltpu
from jax.experimental.pallas import tpu_sc as plsc

BATCH = 16384
NUM_FIELDS = 26
FIELD_SIZE = 100000
L = 16
NW = 32
BPW = BATCH // NW
WORDS = BPW * NUM_FIELDS
IDX_ROWS = WORDS // 128

_mesh = plsc.VectorSubcoreMesh(core_axis_name="c", subcore_axis_name="s")


@functools.partial(
    pl.kernel,
    out_type=jax.ShapeDtypeStruct((BATCH,), jnp.float32),
    mesh=_mesh,
    compiler_params=pltpu.CompilerParams(needs_layout_passes=False),
    scratch_types=[
        pltpu.VMEM((IDX_ROWS, 128), jnp.int32),
        pltpu.VMEM((IDX_ROWS, 128), jnp.float32),
        pltpu.VMEM((BPW,), jnp.float32),
        pltpu.VMEM((L,), jnp.float32),
        pltpu.SemaphoreType.DMA,
    ],
)
def _sc_kernel(idx_hbm, table_hbm, bias_hbm, out_hbm,
               idx_v, rows_v, out_v, bias_v, sem):
    wid = lax.axis_index("s") * 2 + lax.axis_index("c")
    base = wid * BPW

    pltpu.sync_copy(idx_hbm.at[wid], idx_v)
    pltpu.sync_copy(bias_hbm, bias_v)

    for r in range(IDX_ROWS):
        pltpu.async_copy(table_hbm.at[idx_v.at[r]], rows_v.at[r], sem).wait()

    bias_vec = bias_v[...]

    for j in range(BPW // L):
        c = j // 8
        col = (j % 8) * L
        acc = bias_vec
        for f in range(NUM_FIELDS):
            acc = acc + rows_v[f * (BPW // 128) + c, pl.ds(col, L)]
        out_v[pl.ds(j * L, L)] = acc

    pltpu.sync_copy(out_v, out_hbm.at[pl.ds(base, BPW)])


def kernel(x, fc_weight, bias):
    offs = (jnp.arange(NUM_FIELDS, dtype=jnp.int32) * FIELD_SIZE)[None, :]
    idx = (x + offs).T.reshape(NUM_FIELDS, NW, BPW)
    idx = jnp.transpose(idx, (1, 0, 2)).reshape(NW, IDX_ROWS, 128)
    bias16 = jnp.broadcast_to(bias, (L,))
    out = _sc_kernel(idx, fc_weight.reshape(-1), bias16)
    return out.reshape(BATCH, 1)


# speedup vs baseline: 1.2729x; 1.2729x over previous
import functools

import jax
import jax.numpy as jnp
from jax import lax
from jax.experimental import pallas as pl
from jax.experimental.pallas import tpu as pltpu
from jax.experimental.pallas import tpu_sc as plsc

BATCH = 16384
NUM_FIELDS = 26
FIELD_SIZE = 100000
L = 16
NW = 32
BPW = BATCH // NW
WORDS = BPW * NUM_FIELDS
IDX_ROWS = WORDS // 128

_mesh = plsc.VectorSubcoreMesh(core_axis_name="c", subcore_axis_name="s")


@functools.partial(
    pl.kernel,
    out_type=jax.ShapeDtypeStruct((BATCH,), jnp.float32),
    mesh=_mesh,
    compiler_params=pltpu.CompilerParams(needs_layout_passes=False),
    scratch_types=[
        pltpu.VMEM((WORDS,), jnp.int32),
        pltpu.VMEM((IDX_ROWS, 128), jnp.int32),
        pltpu.VMEM((IDX_ROWS, 128), jnp.float32),
        pltpu.VMEM((BPW,), jnp.float32),
        pltpu.VMEM((L,), jnp.float32),
        pltpu.SemaphoreType.DMA,
    ],
)
def _sc_kernel(x_hbm, table_hbm, bias_hbm, out_hbm,
               x_v, idx_v, rows_v, out_v, bias_v, sem):
    wid = lax.axis_index("s") * 2 + lax.axis_index("c")
    base = wid * BPW

    pltpu.sync_copy(x_hbm.at[pl.ds(base * NUM_FIELDS, WORDS)], x_v)
    pltpu.sync_copy(bias_hbm, bias_v)

    lane = lax.iota(jnp.int32, L)

    for r in range(IDX_ROWS):
        f, c = divmod(r, BPW // 128)
        for k in range(8):
            j = c * 8 + k
            src = lane * NUM_FIELDS + (j * L * NUM_FIELDS + f)
            idx_v[r, pl.ds(k * L, L)] = plsc.load_gather(x_v, [src]) \
                + f * FIELD_SIZE
        pltpu.make_async_copy(
            table_hbm.at[idx_v.at[r]], rows_v.at[r], sem).start()

    for r in range(IDX_ROWS):
        pltpu.make_async_copy(
            table_hbm.at[idx_v.at[r]], rows_v.at[r], sem).wait()

    bias_vec = bias_v[...]

    for j in range(BPW // L):
        c = j // 8
        col = (j % 8) * L
        acc = bias_vec
        for f in range(NUM_FIELDS):
            acc = acc + rows_v[f * (BPW // 128) + c, pl.ds(col, L)]
        out_v[pl.ds(j * L, L)] = acc

    pltpu.sync_copy(out_v, out_hbm.at[pl.ds(base, BPW)])


def kernel(x, fc_weight, bias):
    out = _sc_kernel(x.reshape(-1), fc_weight.reshape(-1),
                     jnp.broadcast_to(bias, (L,)))
    return out.reshape(BATCH, 1)


# speedup vs baseline: 1.4301x; 1.1235x over previous
import functools

import jax
import jax.numpy as jnp
from jax import lax
from jax.experimental import pallas as pl
from jax.experimental.pallas import tpu as pltpu
from jax.experimental.pallas import tpu_sc as plsc

BATCH = 16384
NUM_FIELDS = 26
FIELD_SIZE = 100000
L = 16
NW = 32
BPW = BATCH // NW
WORDS = BPW * NUM_FIELDS
IDX_ROWS = WORDS // 128

_mesh = plsc.VectorSubcoreMesh(core_axis_name="c", subcore_axis_name="s")


@functools.partial(
    pl.kernel,
    out_type=jax.ShapeDtypeStruct((BATCH,), jnp.float32),
    mesh=_mesh,
    compiler_params=pltpu.CompilerParams(needs_layout_passes=False),
    scratch_types=[
        pltpu.VMEM((NUM_FIELDS, BPW), jnp.int32),
        pltpu.VMEM((IDX_ROWS, 128), jnp.int32),
        pltpu.VMEM((IDX_ROWS, 128), jnp.float32),
        pltpu.VMEM((BPW,), jnp.float32),
        pltpu.VMEM((L,), jnp.float32),
        pltpu.SemaphoreType.DMA,
    ],
)
def _sc_kernel(x_hbm, table_hbm, bias_hbm, out_hbm,
               x_v, idx_v, rows_v, out_v, bias_v, sem):
    wid = lax.axis_index("s") * 2 + lax.axis_index("c")
    base = wid * BPW

    pltpu.sync_copy(x_hbm.at[:, pl.ds(base, BPW)], x_v)
    pltpu.sync_copy(bias_hbm, bias_v)

    for r in range(IDX_ROWS):
        f, c = divmod(r, BPW // 128)
        for k in range(8):
            col = c * 128 + k * L
            idx_v[r, pl.ds(k * L, L)] = x_v[f, pl.ds(col, L)] \
                + f * FIELD_SIZE
        pltpu.make_async_copy(
            table_hbm.at[idx_v.at[r]], rows_v.at[r], sem).start()

    for r in range(IDX_ROWS):
        pltpu.make_async_copy(
            table_hbm.at[idx_v.at[r]], rows_v.at[r], sem).wait()

    bias_vec = bias_v[...]

    for j in range(BPW // L):
        c = j // 8
        col = (j % 8) * L
        acc = bias_vec
        for f in range(NUM_FIELDS):
            acc = acc + rows_v[f * (BPW // 128) + c, pl.ds(col, L)]
        out_v[pl.ds(j * L, L)] = acc

    pltpu.sync_copy(out_v, out_hbm.at[pl.ds(base, BPW)])


def kernel(x, fc_weight, bias):
    out = _sc_kernel(x.T, fc_weight.reshape(-1),
                     jnp.broadcast_to(bias, (L,)))
    return out.reshape(BATCH, 1)
